# baseline (speedup 1.0000x reference)
import functools

import numpy as np
import jax
import jax.numpy as jnp
from jax import lax
from jax.experimental import pallas as pl
from jax.experimental.pallas import tpu as pltpu
from jax.experimental.pallas import tpu_sc as plsc

_B, _L, _D = 4096, 200, 64
_NW = 32
_NB = _B // _NW
_LC = 2
_NCH = _L // _LC
_QW = 4 * _LC
_NQ = _L // _QW
_DT, _DL = 8, 8
_BT = _NW


def _pos_enc_np():
    pe = np.array(
        [[pos / np.power(10000, 2 * i / _D) for i in range(_D)] for pos in range(_L)],
        dtype=np.float32,
    )
    pe[:, 0::2] = np.sin(pe[:, 0::2])
    pe[:, 1::2] = np.cos(pe[:, 1::2])
    return pe


_MESH = plsc.VectorSubcoreMesh(core_axis_name="c", subcore_axis_name="s")


@functools.partial(
    pl.kernel,
    mesh=_MESH,
    compiler_params=pltpu.CompilerParams(use_tc_tiling_on_sc=False, needs_layout_passes=False),
    out_type=jax.ShapeDtypeStruct((_L, _DT, _BT, _DL, _NB), jnp.float32),
    scratch_types=[
        pltpu.VMEM((2 * _NB, _QW), jnp.int32),
        pltpu.VMEM((2, _LC, _NB), jnp.int32),
        pltpu.VMEM((2, _LC, _NB, _D), jnp.float32),
        pltpu.VMEM((2, _LC, _DT, 1, _DL, _NB), jnp.float32),
        pltpu.VMEM((_L, _D), jnp.float32),
        pltpu.SemaphoreType.DMA,
        pltpu.SemaphoreType.DMA,
        pltpu.SemaphoreType.DMA,
    ],
)
def _embed_sc(ids_hbm, pe_hbm, table_hbm, phys_hbm,
              sb, idx_t, g_v, tblk, pe_v, sem_i, sem_g, sem_o):
    wid = lax.axis_index("s") * 2 + lax.axis_index("c")
    b0 = wid * _NB
    iota16 = lax.iota(jnp.int32, 16)
    pltpu.sync_copy(pe_hbm, pe_v)

    def slab_copy(q):
        return pltpu.make_async_copy(
            ids_hbm.at[pl.ds(b0, _NB), pl.ds(q * _QW, _QW)],
            sb.at[pl.ds(lax.rem(q, 2) * _NB, _NB)],
            sem_i,
        )

    def build_idx(c, e):
        srow = lax.rem(c // 4, 2) * _NB
        col0 = lax.rem(c, 4) * _LC
        for lj in range(_LC):
            col = jnp.full((16,), 0, jnp.int32) + (col0 + lj)
            for blg in range(_NB // 16):
                v = plsc.load_gather(sb, [srow + blg * 16 + iota16, col])
                idx_t[e, lj, pl.ds(blg * 16, 16)] = v

    def gather_copy(e, lj):
        return pltpu.make_async_copy(
            table_hbm.at[idx_t.at[e, lj]], g_v.at[e, lj], sem_g)

    def out_copy(c, e):
        return pltpu.make_async_copy(
            tblk.at[e],
            phys_hbm.at[pl.ds(c * _LC, _LC), pl.ds(0, _DT), pl.ds(wid, 1)],
            sem_o,
        )

    def transpose_add(c, e):
        l0 = c * _LC
        sple = jnp.full((16,), 0, jnp.int32) + e
        for lj in range(_LC):
            splj = jnp.full((16,), lj, jnp.int32)

            spll = jnp.full((16,), 0, jnp.int32) + (l0 + lj)

            def dt_body(dt, carry):
                for dl in range(_DL):
                    d = dt * _DL + dl
                    spld = jnp.full((16,), 0, jnp.int32) + d
                    p = plsc.load_gather(pe_v, [spll, spld])
                    vs = [
                        plsc.load_gather(
                            g_v, [sple, splj, blg * 16 + iota16, spld])
                        for blg in range(_NB // 16)
                    ]
                    for blg in range(_NB // 16):
                        tblk[e, lj, dt, 0, dl, pl.ds(blg * 16, 16)] = vs[blg] + p
                return carry

            lax.fori_loop(0, _DT, dt_body, 0)

    cp = slab_copy(0)
    cp.start()
    cp.wait()
    build_idx(0, 0)
    for lj in range(_LC):
        gather_copy(0, lj).start()

    def pair_body(p, carry):
        for e in range(2):
            c = 2 * p + e
            for lj in range(_LC):
                gather_copy(e, lj).wait()

            @pl.when(c + 1 < _NCH)
            def _feed_next():
                if e == 1:
                    @pl.when(lax.rem(p, 2) == 1)
                    def _wait_slab():
                        slab_copy((c + 1) // 4).wait()
                build_idx(c + 1, 1 - e)
                for lj in range(_LC):
                    gather_copy(1 - e, lj).start()

            if e == 0:
                @pl.when(lax.rem(p, 2) == 0)
                def _prefetch_slab():
                    @pl.when(p // 2 + 1 < _NQ)
                    def _go():
                        slab_copy(p // 2 + 1).start()

            @pl.when(c >= 2)
            def _tblk_free():
                out_copy(c - 2, e).wait()

            transpose_add(c, e)
            out_copy(c, e).start()
        return carry

    lax.fori_loop(0, _NCH // 2, pair_body, 0)
    out_copy(_NCH - 2, 0).wait()
    out_copy(_NCH - 1, 1).wait()


def kernel(ids, table):
    pe = jnp.asarray(_pos_enc_np())
    phys = _embed_sc(ids.astype(jnp.int32), pe, table)
    return phys.transpose(2, 4, 0, 1, 3).reshape(_B, _L, _D)


# speedup vs baseline: 1.2235x; 1.2235x over previous
import functools

import numpy as np
import jax
import jax.numpy as jnp
from jax import lax
from jax.experimental import pallas as pl
from jax.experimental.pallas import tpu as pltpu
from jax.experimental.pallas import tpu_sc as plsc

_B, _L, _D = 4096, 200, 64
_NW = 32
_NB = _B // _NW
_H = 104
_OFF = (0, 96)
_GROUPS = _NB // 2
_NBUF = 8


def _pos_enc_np():
    pe = np.array(
        [[pos / np.power(10000, 2 * i / _D) for i in range(_D)] for pos in range(_L)],
        dtype=np.float32,
    )
    pe[:, 0::2] = np.sin(pe[:, 0::2])
    pe[:, 1::2] = np.cos(pe[:, 1::2])
    return pe


_MESH = plsc.VectorSubcoreMesh(core_axis_name="c", subcore_axis_name="s")


@functools.partial(
    pl.kernel,
    mesh=_MESH,
    compiler_params=pltpu.CompilerParams(use_tc_tiling_on_sc=False),
    out_type=jax.ShapeDtypeStruct((_B, _L, _D), jnp.float32),
    scratch_types=[
        pltpu.VMEM((_NB, _H), jnp.int32),
        pltpu.VMEM((_NB, _H), jnp.int32),
        pltpu.VMEM((_NBUF, _H, _D), jnp.float32),
        pltpu.VMEM((_L, _D), jnp.float32),
        pltpu.SemaphoreType.DMA,
        pltpu.SemaphoreType.DMA,
    ],
)
def _embed_sc(ids_hbm, pe_hbm, table_hbm, out_hbm,
              idx_a, idx_b, rows_v, pe_v, sem_g, sem_o):
    wid = lax.axis_index("s") * 2 + lax.axis_index("c")
    b0 = wid * _NB
    pltpu.sync_copy(pe_hbm, pe_v)
    pltpu.sync_copy(ids_hbm.at[pl.ds(b0, _NB), pl.ds(_OFF[0], _H)], idx_a)
    pltpu.sync_copy(ids_hbm.at[pl.ds(b0, _NB), pl.ds(_OFF[1], _H)], idx_b)

    def unit_idx(b, half):
        return idx_a.at[b] if half == 0 else idx_b.at[b]

    def start_group_gathers(g, base):
        for k in range(4):
            pltpu.async_copy(
                table_hbm.at[unit_idx(2 * g + k // 2, k % 2)],
                rows_v.at[base + k],
                sem_g,
            )

    def wait_out_one():
        pltpu.make_async_copy(
            rows_v.at[0], out_hbm.at[b0, pl.ds(0, _H)], sem_o
        ).wait()

    start_group_gathers(0, 0)

    def pair_body(gg, carry):
        for parity in range(2):
            g = 2 * gg + parity
            base = parity * 4
            for k in range(4):
                pltpu.make_async_copy(
                    table_hbm.at[unit_idx(2 * g + k // 2, k % 2)],
                    rows_v.at[base + k],
                    sem_g,
                ).wait()

            def add_body(i, c2):
                for jj in range(_D // 16):
                    sl = pl.ds(jj * 16, 16)
                    pea = pe_v[_OFF[0] + i, sl]
                    peb = pe_v[_OFF[1] + i, sl]
                    rows_v[base + 0, i, sl] = rows_v[base + 0, i, sl] + pea
                    rows_v[base + 2, i, sl] = rows_v[base + 2, i, sl] + pea
                    rows_v[base + 1, i, sl] = rows_v[base + 1, i, sl] + peb
                    rows_v[base + 3, i, sl] = rows_v[base + 3, i, sl] + peb
                return c2

            lax.fori_loop(0, _H, add_body, 0, unroll=2)

            for k in range(4):
                pltpu.async_copy(
                    rows_v.at[base + k],
                    out_hbm.at[b0 + 2 * g + k // 2, pl.ds(_OFF[k % 2], _H)],
                    sem_o,
                )

            @pl.when(g >= 1)
            def _wait_prev_outs():
                for _ in range(4):
                    wait_out_one()

            @pl.when(g + 1 < _GROUPS)
            def _start_next():
                start_group_gathers(g + 1, 4 - base)
        return carry

    lax.fori_loop(0, _GROUPS // 2, pair_body, 0)
    for _ in range(4):
        wait_out_one()


def kernel(ids, table):
    pe = jnp.asarray(_pos_enc_np())
    return _embed_sc(ids.astype(jnp.int32), pe, table)


# speedup vs baseline: 1.2963x; 1.0595x over previous
import functools

import numpy as np
import jax
import jax.numpy as jnp
from jax import lax
from jax.experimental import pallas as pl
from jax.experimental.pallas import tpu as pltpu
from jax.experimental.pallas import tpu_sc as plsc

_B, _L, _D = 4096, 200, 64
_NW = 32
_NB = _B // _NW
_H = 104
_OFF = (0, 96)
_GROUPS = _NB // 2
_NBUF = 8


def _pos_enc_np():
    pe = np.array(
        [[pos / np.power(10000, 2 * i / _D) for i in range(_D)] for pos in range(_L)],
        dtype=np.float32,
    )
    pe[:, 0::2] = np.sin(pe[:, 0::2])
    pe[:, 1::2] = np.cos(pe[:, 1::2])
    return pe


_MESH = plsc.VectorSubcoreMesh(core_axis_name="c", subcore_axis_name="s")


@functools.partial(
    pl.kernel,
    mesh=_MESH,
    compiler_params=pltpu.CompilerParams(use_tc_tiling_on_sc=False),
    out_type=jax.ShapeDtypeStruct((_B, _L, _D), jnp.float32),
    scratch_types=[
        pltpu.VMEM((_NB, _H), jnp.int32),
        pltpu.VMEM((_NB, _H), jnp.int32),
        pltpu.VMEM((_NBUF, _H, _D), jnp.float32),
        pltpu.VMEM((_L, _D), jnp.float32),
        pltpu.SemaphoreType.DMA,
        pltpu.SemaphoreType.DMA,
    ],
)
def _embed_sc(ids_hbm, pe_hbm, table_hbm, out_hbm,
              idx_a, idx_b, rows_v, pe_v, sem_g, sem_o):
    wid = lax.axis_index("s") * 2 + lax.axis_index("c")
    b0 = wid * _NB
    pltpu.sync_copy(pe_hbm, pe_v)
    pltpu.sync_copy(ids_hbm.at[pl.ds(b0, _NB), pl.ds(_OFF[0], _H)], idx_a)
    pltpu.sync_copy(ids_hbm.at[pl.ds(b0, _NB), pl.ds(_OFF[1], _H)], idx_b)

    def unit_idx(b, half):
        return idx_a.at[b] if half == 0 else idx_b.at[b]

    def start_group_gathers(g, base):
        for k in range(4):
            pltpu.async_copy(
                table_hbm.at[unit_idx(2 * g + k // 2, k % 2)],
                rows_v.at[base + k],
                sem_g,
            )

    def wait_out_one():
        pltpu.make_async_copy(
            rows_v.at[0], out_hbm.at[b0, pl.ds(0, _H)], sem_o
        ).wait()

    start_group_gathers(0, 0)

    def pair_body(gg, carry):
        for parity in range(2):
            g = 2 * gg + parity
            base = parity * 4
            for k in range(4):
                pltpu.make_async_copy(
                    table_hbm.at[unit_idx(2 * g + k // 2, k % 2)],
                    rows_v.at[base + k],
                    sem_g,
                ).wait()

            @pl.when(g >= 1)
            def _wait_prev_outs():
                for _ in range(4):
                    wait_out_one()

            @pl.when(g + 1 < _GROUPS)
            def _start_next():
                start_group_gathers(g + 1, 4 - base)

            def add_body(i, c2):
                for jj in range(_D // 16):
                    sl = pl.ds(jj * 16, 16)
                    pea = pe_v[_OFF[0] + i, sl]
                    peb = pe_v[_OFF[1] + i, sl]
                    rows_v[base + 0, i, sl] = rows_v[base + 0, i, sl] + pea
                    rows_v[base + 2, i, sl] = rows_v[base + 2, i, sl] + pea
                    rows_v[base + 1, i, sl] = rows_v[base + 1, i, sl] + peb
                    rows_v[base + 3, i, sl] = rows_v[base + 3, i, sl] + peb
                return c2

            lax.fori_loop(0, _H, add_body, 0, unroll=2)

            for k in range(4):
                pltpu.async_copy(
                    rows_v.at[base + k],
                    out_hbm.at[b0 + 2 * g + k // 2, pl.ds(_OFF[k % 2], _H)],
                    sem_o,
                )
        return carry

    lax.fori_loop(0, _GROUPS // 2, pair_body, 0)
    for _ in range(4):
        wait_out_one()


def kernel(ids, table):
    pe = jnp.asarray(_pos_enc_np())
    return _embed_sc(ids.astype(jnp.int32), pe, table)
